# baseline (speedup 1.0000x reference)
import math

import jax
import jax.numpy as jnp
from jax.experimental import pallas as pl
from jax.experimental.pallas import tpu as pltpu

_B, _S, _D = 8, 4096, 1024
_K = 64
_H = 256
_SF = 128
_SH = 256

_T1 = 2048
_NS1 = _S // _T1
_T2 = 8192
_NT2 = (_K * _D) // _T2

_SQRT2_INV = 0.7071067811865476
_LOG1E4 = math.log(10000.0)


def _pool_copy_body(seq_ref, emb_ref, out_ref, acc_ref):
    b = pl.program_id(0)
    s = pl.program_id(1)
    x = emb_ref[0]
    out_ref[0] = x
    seqlen = seq_ref[b]
    rows = s * _T1 + jax.lax.broadcasted_iota(jnp.int32, (_T1, 1), 0)
    mask = (rows < seqlen).astype(jnp.float32)
    partial = jnp.sum(x * mask, axis=0)[None, :]

    @pl.when(s == 0)
    def _():
        acc_ref[0] = partial

    @pl.when(s != 0)
    def _():
        acc_ref[0] = acc_ref[0] + partial


def _postfix_body(pooled_ref, seqf_ref, t_ref, W1_ref, b1_ref, Ws1_ref,
                  bs1_ref, W2_ref, b2_ref, Ws2_ref, bs2_ref, pf_ref,
                  h_scr, hs_scr):
    t = pl.program_id(0)

    @pl.when(t == 0)
    def _():
        denom = jnp.maximum(seqf_ref[...], 1.0)
        pooled = pooled_ref[:, 0, :] / denom
        z = jnp.dot(pooled, W1_ref[...],
                    preferred_element_type=jnp.float32,
                    precision=jax.lax.Precision.HIGHEST) + b1_ref[...]
        h_scr[...] = 0.5 * z * (1.0 + jax.lax.erf(z * _SQRT2_INV))
        half = _SF // 2
        k_iota = jax.lax.broadcasted_iota(
            jnp.int32, (1, half), 1).astype(jnp.float32)
        freqs = jnp.exp(-(_LOG1E4 / half) * k_iota)
        angles = t_ref[...] * freqs
        sigma = jnp.concatenate([jnp.cos(angles), jnp.sin(angles)], axis=1)
        zs = jnp.dot(sigma, Ws1_ref[...],
                     preferred_element_type=jnp.float32,
                     precision=jax.lax.Precision.HIGHEST) + bs1_ref[...]
        hs_scr[...] = zs * jax.nn.sigmoid(zs)

    pf = jnp.dot(h_scr[...], W2_ref[...],
                 preferred_element_type=jnp.float32,
                 precision=jax.lax.Precision.HIGHEST)
    pf = pf + jnp.dot(hs_scr[...], Ws2_ref[...],
                      preferred_element_type=jnp.float32,
                      precision=jax.lax.Precision.HIGHEST)
    pf_ref[...] = pf + b2_ref[...] + bs2_ref[...]


def _splice_body(seq_ref, src_ref, pf_ref, out_ref):
    b = pl.program_id(0)
    j = pl.program_id(1)
    seqlen = seq_ref[b]
    r = jax.lax.rem(seqlen, _K)
    pf = pf_ref[0]
    rolled = pltpu.roll(pf, r, 0)
    rows = jax.lax.broadcasted_iota(jnp.int32, (_K, 1), 0)
    is_first = (j == 0)
    keep_new = ((rows >= r) & is_first) | ((rows < r) & jnp.logical_not(is_first))
    out_ref[0] = jnp.where(keep_new, rolled, src_ref[0])


def kernel(crossattn_emb, crossattn_seqlens, timesteps, W1, b1, W2, b2,
           Ws1, bs1, Ws2, bs2):
    seq_i32 = crossattn_seqlens.astype(jnp.int32)

    grid1 = pltpu.PrefetchScalarGridSpec(
        num_scalar_prefetch=1,
        grid=(_B, _NS1),
        in_specs=[pl.BlockSpec((1, _T1, _D), lambda b, s, seq: (b, s, 0))],
        out_specs=[
            pl.BlockSpec((1, _T1, _D), lambda b, s, seq: (b, s, 0)),
            pl.BlockSpec((1, 1, _D), lambda b, s, seq: (b, 0, 0)),
        ],
    )
    out1, pooled_sum = pl.pallas_call(
        _pool_copy_body,
        grid_spec=grid1,
        out_shape=[
            jax.ShapeDtypeStruct((_B, _S, _D), jnp.float32),
            jax.ShapeDtypeStruct((_B, 1, _D), jnp.float32),
        ],
        compiler_params=pltpu.CompilerParams(
            dimension_semantics=("arbitrary", "arbitrary"),
            vmem_limit_bytes=100 * 1024 * 1024),
    )(seq_i32, crossattn_emb)

    seqf = seq_i32.astype(jnp.float32).reshape(_B, 1)
    t2d = timesteps.astype(jnp.float32).reshape(_B, 1)
    pf = pl.pallas_call(
        _postfix_body,
        grid=(_NT2,),
        in_specs=[
            pl.BlockSpec((_B, 1, _D), lambda t: (0, 0, 0)),
            pl.BlockSpec((_B, 1), lambda t: (0, 0)),
            pl.BlockSpec((_B, 1), lambda t: (0, 0)),
            pl.BlockSpec((_D, _H), lambda t: (0, 0)),
            pl.BlockSpec((1, _H), lambda t: (0, 0)),
            pl.BlockSpec((_SF, _SH), lambda t: (0, 0)),
            pl.BlockSpec((1, _SH), lambda t: (0, 0)),
            pl.BlockSpec((_H, _T2), lambda t: (0, t)),
            pl.BlockSpec((1, _T2), lambda t: (0, t)),
            pl.BlockSpec((_SH, _T2), lambda t: (0, t)),
            pl.BlockSpec((1, _T2), lambda t: (0, t)),
        ],
        out_specs=pl.BlockSpec((_B, _T2), lambda t: (0, t)),
        out_shape=jax.ShapeDtypeStruct((_B, _K * _D), jnp.float32),
        scratch_shapes=[
            pltpu.VMEM((_B, _H), jnp.float32),
            pltpu.VMEM((_B, _SH), jnp.float32),
        ],
        compiler_params=pltpu.CompilerParams(
            dimension_semantics=("arbitrary",),
            vmem_limit_bytes=100 * 1024 * 1024),
    )(pooled_sum, seqf, t2d, W1, b1.reshape(1, _H), Ws1, bs1.reshape(1, _SH),
      W2, b2.reshape(1, _K * _D), Ws2, bs2.reshape(1, _K * _D))
    pf3 = pf.reshape(_B, _K, _D)

    grid4 = pltpu.PrefetchScalarGridSpec(
        num_scalar_prefetch=1,
        grid=(_B, 2),
        in_specs=[
            pl.BlockSpec((1, _K, _D), lambda b, j, seq: (b, seq[b] // _K + j, 0)),
            pl.BlockSpec((1, _K, _D), lambda b, j, seq: (b, 0, 0)),
        ],
        out_specs=pl.BlockSpec((1, _K, _D), lambda b, j, seq: (b, seq[b] // _K + j, 0)),
    )
    out = pl.pallas_call(
        _splice_body,
        grid_spec=grid4,
        out_shape=jax.ShapeDtypeStruct((_B, _S, _D), jnp.float32),
        input_output_aliases={1: 0},
        compiler_params=pltpu.CompilerParams(
            dimension_semantics=("arbitrary", "arbitrary")),
    )(seq_i32, out1, pf3)
    return out


# speedup vs baseline: 1.0012x; 1.0012x over previous
import math

import jax
import jax.numpy as jnp
from jax.experimental import pallas as pl
from jax.experimental.pallas import tpu as pltpu

_B, _S, _D = 8, 4096, 1024
_K = 64
_H = 256
_SF = 128
_SH = 256

_T1 = 2048
_NS1 = _S // _T1
_T2 = 8192
_NT2 = (_K * _D) // _T2

_SQRT2_INV = 0.7071067811865476
_LOG1E4 = math.log(10000.0)


def _pool_copy_body(seq_ref, emb_ref, out_ref, acc_ref):
    b = pl.program_id(0)
    s = pl.program_id(1)
    x = emb_ref[0]
    out_ref[0] = x
    seqlen = seq_ref[b]
    rows = s * _T1 + jax.lax.broadcasted_iota(jnp.int32, (_T1, 1), 0)
    mask = (rows < seqlen).astype(jnp.float32)
    partial = jnp.sum(x * mask, axis=0)[None, :]

    @pl.when(s == 0)
    def _():
        acc_ref[0] = partial

    @pl.when(s != 0)
    def _():
        acc_ref[0] = acc_ref[0] + partial


def _postfix_body(pooled_ref, seqf_ref, t_ref, W1_ref, b1_ref, Ws1_ref,
                  bs1_ref, W2_ref, b2_ref, Ws2_ref, bs2_ref, pf_ref,
                  h_scr, hs_scr):
    t = pl.program_id(0)

    @pl.when(t == 0)
    def _():
        denom = jnp.maximum(seqf_ref[...], 1.0)
        pooled = pooled_ref[:, 0, :] / denom
        z = jnp.dot(pooled, W1_ref[...],
                    preferred_element_type=jnp.float32,
                    precision=jax.lax.Precision.HIGHEST) + b1_ref[...]
        h_scr[...] = 0.5 * z * (1.0 + jax.lax.erf(z * _SQRT2_INV))
        half = _SF // 2
        k_iota = jax.lax.broadcasted_iota(
            jnp.int32, (1, half), 1).astype(jnp.float32)
        freqs = jnp.exp(-(_LOG1E4 / half) * k_iota)
        angles = t_ref[...] * freqs
        sigma = jnp.concatenate([jnp.cos(angles), jnp.sin(angles)], axis=1)
        zs = jnp.dot(sigma, Ws1_ref[...],
                     preferred_element_type=jnp.float32,
                     precision=jax.lax.Precision.HIGHEST) + bs1_ref[...]
        hs_scr[...] = zs * jax.nn.sigmoid(zs)

    pf = jnp.dot(h_scr[...], W2_ref[...],
                 preferred_element_type=jnp.float32,
                 precision=jax.lax.Precision.HIGHEST)
    pf = pf + jnp.dot(hs_scr[...], Ws2_ref[...],
                      preferred_element_type=jnp.float32,
                      precision=jax.lax.Precision.HIGHEST)
    pf_ref[...] = pf + b2_ref[...] + bs2_ref[...]


def _splice_body(seq_ref, src_ref, pf_ref, out_ref):
    b = pl.program_id(0)
    j = pl.program_id(1)
    seqlen = seq_ref[b]
    r = jax.lax.rem(seqlen, _K)
    pf = pf_ref[0]
    rolled = pltpu.roll(pf, r, 0)
    rows = jax.lax.broadcasted_iota(jnp.int32, (_K, 1), 0)
    is_first = (j == 0)
    keep_new = ((rows >= r) & is_first) | ((rows < r) & jnp.logical_not(is_first))
    out_ref[0] = jnp.where(keep_new, rolled, src_ref[0])


def kernel(crossattn_emb, crossattn_seqlens, timesteps, W1, b1, W2, b2,
           Ws1, bs1, Ws2, bs2):
    seq_i32 = crossattn_seqlens.astype(jnp.int32)

    grid1 = pltpu.PrefetchScalarGridSpec(
        num_scalar_prefetch=1,
        grid=(_B, _NS1),
        in_specs=[pl.BlockSpec((1, _T1, _D), lambda b, s, seq: (b, s, 0))],
        out_specs=[
            pl.BlockSpec((1, _T1, _D), lambda b, s, seq: (b, s, 0)),
            pl.BlockSpec((1, 1, _D), lambda b, s, seq: (b, 0, 0)),
        ],
    )
    out1, pooled_sum = pl.pallas_call(
        _pool_copy_body,
        grid_spec=grid1,
        out_shape=[
            jax.ShapeDtypeStruct((_B, _S, _D), jnp.float32),
            jax.ShapeDtypeStruct((_B, 1, _D), jnp.float32),
        ],
        compiler_params=pltpu.CompilerParams(
            dimension_semantics=("arbitrary", "arbitrary")),
    )(seq_i32, crossattn_emb)

    seqf = seq_i32.astype(jnp.float32).reshape(_B, 1)
    t2d = timesteps.astype(jnp.float32).reshape(_B, 1)
    pf = pl.pallas_call(
        _postfix_body,
        grid=(_NT2,),
        in_specs=[
            pl.BlockSpec((_B, 1, _D), lambda t: (0, 0, 0)),
            pl.BlockSpec((_B, 1), lambda t: (0, 0)),
            pl.BlockSpec((_B, 1), lambda t: (0, 0)),
            pl.BlockSpec((_D, _H), lambda t: (0, 0)),
            pl.BlockSpec((1, _H), lambda t: (0, 0)),
            pl.BlockSpec((_SF, _SH), lambda t: (0, 0)),
            pl.BlockSpec((1, _SH), lambda t: (0, 0)),
            pl.BlockSpec((_H, _T2), lambda t: (0, t)),
            pl.BlockSpec((1, _T2), lambda t: (0, t)),
            pl.BlockSpec((_SH, _T2), lambda t: (0, t)),
            pl.BlockSpec((1, _T2), lambda t: (0, t)),
        ],
        out_specs=pl.BlockSpec((_B, _T2), lambda t: (0, t)),
        out_shape=jax.ShapeDtypeStruct((_B, _K * _D), jnp.float32),
        scratch_shapes=[
            pltpu.VMEM((_B, _H), jnp.float32),
            pltpu.VMEM((_B, _SH), jnp.float32),
        ],
        compiler_params=pltpu.CompilerParams(
            dimension_semantics=("arbitrary",)),
    )(pooled_sum, seqf, t2d, W1, b1.reshape(1, _H), Ws1, bs1.reshape(1, _SH),
      W2, b2.reshape(1, _K * _D), Ws2, bs2.reshape(1, _K * _D))
    pf3 = pf.reshape(_B, _K, _D)

    grid4 = pltpu.PrefetchScalarGridSpec(
        num_scalar_prefetch=1,
        grid=(_B, 2),
        in_specs=[
            pl.BlockSpec((1, _K, _D), lambda b, j, seq: (b, seq[b] // _K + j, 0)),
            pl.BlockSpec((1, _K, _D), lambda b, j, seq: (b, 0, 0)),
        ],
        out_specs=pl.BlockSpec((1, _K, _D), lambda b, j, seq: (b, seq[b] // _K + j, 0)),
    )
    out = pl.pallas_call(
        _splice_body,
        grid_spec=grid4,
        out_shape=jax.ShapeDtypeStruct((_B, _S, _D), jnp.float32),
        input_output_aliases={1: 0},
        compiler_params=pltpu.CompilerParams(
            dimension_semantics=("arbitrary", "arbitrary")),
    )(seq_i32, out1, pf3)
    return out


# speedup vs baseline: 1.0272x; 1.0260x over previous
import math

import jax
import jax.numpy as jnp
from jax.experimental import pallas as pl
from jax.experimental.pallas import tpu as pltpu

_B, _S, _D = 8, 4096, 1024
_K = 64
_H = 256
_SF = 128
_SH = 256

_T1 = 2048
_NS1 = _S // _T1
_T2 = 8192
_NT2 = (_K * _D) // _T2

_SQRT2_INV = 0.7071067811865476
_LOG1E4 = math.log(10000.0)


def _pool_copy_body(seq_ref, emb_ref, out_ref, acc_ref):
    b = pl.program_id(0)
    s = pl.program_id(1)
    x = emb_ref[0]
    out_ref[0] = x
    seqlen = seq_ref[b]
    rows = s * _T1 + jax.lax.broadcasted_iota(jnp.int32, (_T1, 1), 0)
    mask = (rows < seqlen).astype(jnp.float32)
    partial = jnp.sum(x * mask, axis=0)[None, :]

    @pl.when(s == 0)
    def _():
        acc_ref[0] = partial

    @pl.when(s != 0)
    def _():
        acc_ref[0] = acc_ref[0] + partial


def _postfix_splice_body(seq_ref, pooled_ref, seqf_ref, t_ref, W1_ref, b1_ref,
                         Ws1_ref, bs1_ref, W2_ref, b2_ref, Ws2_ref, bs2_ref,
                         src_ref, out_ref, h_scr, hs_scr, pf_scr):
    p = pl.program_id(0)

    @pl.when(p == 0)
    def _():
        denom = jnp.maximum(seqf_ref[...], 1.0)
        pooled = pooled_ref[:, 0, :] / denom
        z = jnp.dot(pooled, W1_ref[...],
                    preferred_element_type=jnp.float32,
                    precision=jax.lax.Precision.HIGHEST) + b1_ref[...]
        h_scr[...] = 0.5 * z * (1.0 + jax.lax.erf(z * _SQRT2_INV))
        half = _SF // 2
        k_iota = jax.lax.broadcasted_iota(
            jnp.int32, (1, half), 1).astype(jnp.float32)
        freqs = jnp.exp(-(_LOG1E4 / half) * k_iota)
        angles = t_ref[...] * freqs
        sigma = jnp.concatenate([jnp.cos(angles), jnp.sin(angles)], axis=1)
        zs = jnp.dot(sigma, Ws1_ref[...],
                     preferred_element_type=jnp.float32,
                     precision=jax.lax.Precision.HIGHEST) + bs1_ref[...]
        hs_scr[...] = zs * jax.nn.sigmoid(zs)

    @pl.when(p < _NT2)
    def _():
        pf = jnp.dot(h_scr[...], W2_ref[...],
                     preferred_element_type=jnp.float32,
                     precision=jax.lax.Precision.HIGHEST)
        pf = pf + jnp.dot(hs_scr[...], Ws2_ref[...],
                          preferred_element_type=jnp.float32,
                          precision=jax.lax.Precision.HIGHEST)
        pf = pf + b2_ref[...] + bs2_ref[...]
        kpb = _T2 // _D
        t = jnp.minimum(p, _NT2 - 1)
        pf_scr[:, pl.ds(t * kpb, kpb), :] = pf.reshape(_B, kpb, _D)
        out_ref[0] = src_ref[0]

    @pl.when(p >= _NT2)
    def _():
        q = p - _NT2
        b = q // 2
        j = q - b * 2
        seqlen = seq_ref[b]
        r = jax.lax.rem(seqlen, _K)
        pf_b = pf_scr[b]
        rolled = pltpu.roll(pf_b, r, 0)
        rows = jax.lax.broadcasted_iota(jnp.int32, (_K, 1), 0)
        is_first = (j == 0)
        keep_new = (((rows >= r) & is_first)
                    | ((rows < r) & jnp.logical_not(is_first)))
        out_ref[0] = jnp.where(keep_new, rolled, src_ref[0])


def kernel(crossattn_emb, crossattn_seqlens, timesteps, W1, b1, W2, b2,
           Ws1, bs1, Ws2, bs2):
    seq_i32 = crossattn_seqlens.astype(jnp.int32)

    grid1 = pltpu.PrefetchScalarGridSpec(
        num_scalar_prefetch=1,
        grid=(_B, _NS1),
        in_specs=[pl.BlockSpec((1, _T1, _D), lambda b, s, seq: (b, s, 0))],
        out_specs=[
            pl.BlockSpec((1, _T1, _D), lambda b, s, seq: (b, s, 0)),
            pl.BlockSpec((1, 1, _D), lambda b, s, seq: (b, 0, 0)),
        ],
    )
    out1, pooled_sum = pl.pallas_call(
        _pool_copy_body,
        grid_spec=grid1,
        out_shape=[
            jax.ShapeDtypeStruct((_B, _S, _D), jnp.float32),
            jax.ShapeDtypeStruct((_B, 1, _D), jnp.float32),
        ],
        compiler_params=pltpu.CompilerParams(
            dimension_semantics=("arbitrary", "arbitrary")),
    )(seq_i32, crossattn_emb)

    seqf = seq_i32.astype(jnp.float32).reshape(_B, 1)
    t2d = timesteps.astype(jnp.float32).reshape(_B, 1)

    def _spliced_map(p, seq):
        q = jnp.maximum(p - _NT2, 0)
        b = q // 2
        j = q - b * 2
        return (b, seq[b] // _K + j, 0)

    def _wmap(p, seq):
        return (0, jnp.minimum(p, _NT2 - 1))

    grid2 = pltpu.PrefetchScalarGridSpec(
        num_scalar_prefetch=1,
        grid=(_NT2 + 2 * _B,),
        in_specs=[
            pl.BlockSpec((_B, 1, _D), lambda p, seq: (0, 0, 0)),
            pl.BlockSpec((_B, 1), lambda p, seq: (0, 0)),
            pl.BlockSpec((_B, 1), lambda p, seq: (0, 0)),
            pl.BlockSpec((_D, _H), lambda p, seq: (0, 0)),
            pl.BlockSpec((1, _H), lambda p, seq: (0, 0)),
            pl.BlockSpec((_SF, _SH), lambda p, seq: (0, 0)),
            pl.BlockSpec((1, _SH), lambda p, seq: (0, 0)),
            pl.BlockSpec((_H, _T2), _wmap),
            pl.BlockSpec((1, _T2), _wmap),
            pl.BlockSpec((_SH, _T2), _wmap),
            pl.BlockSpec((1, _T2), _wmap),
            pl.BlockSpec((1, _K, _D), _spliced_map),
        ],
        out_specs=pl.BlockSpec((1, _K, _D), _spliced_map),
        scratch_shapes=[
            pltpu.VMEM((_B, _H), jnp.float32),
            pltpu.VMEM((_B, _SH), jnp.float32),
            pltpu.VMEM((_B, _K, _D), jnp.float32),
        ],
    )
    out = pl.pallas_call(
        _postfix_splice_body,
        grid_spec=grid2,
        out_shape=jax.ShapeDtypeStruct((_B, _S, _D), jnp.float32),
        input_output_aliases={12: 0},
        compiler_params=pltpu.CompilerParams(
            dimension_semantics=("arbitrary",)),
    )(seq_i32, pooled_sum, seqf, t2d, W1, b1.reshape(1, _H), Ws1,
      bs1.reshape(1, _SH), W2, b2.reshape(1, _K * _D), Ws2,
      bs2.reshape(1, _K * _D), out1)
    return out
